# baseline (speedup 1.0000x reference)
import functools

import jax
import jax.numpy as jnp
from jax import lax
from jax.experimental import pallas as pl
from jax.experimental.pallas import tpu as pltpu
from jax.experimental.pallas import tpu_sc as plsc

N = 10000
E = 320000
D = 128
H = 128
ED = 16
L = 5
G = 64

NC = 2
NS = 16
NW = NC * NS
CH = 128
CPW = 80
E_PAD = NW * CPW * CH
NP = 10240
GP = 128

_mesh = plsc.VectorSubcoreMesh(core_axis_name="c", subcore_axis_name="s")


@functools.partial(
    pl.kernel,
    mesh=_mesh,
    out_type=jax.ShapeDtypeStruct((NC, NP, H), jnp.float32),
    scratch_types=[
        pltpu.VMEM((8, CH), jnp.int32),
        pltpu.VMEM((8, CH), jnp.int32),
        pltpu.VMEM((CH, H), jnp.float32),
        pltpu.VMEM((CH // 2, H), jnp.int32),
        pltpu.VMEM_SHARED((NP, H), jnp.float32),
        pltpu.SemaphoreType.DMA,
        pltpu.SemaphoreType.DMA,
    ],
)
def _edge_sc(h_hbm, src_hbm, dst_hbm, e_hbm, out_hbm,
             src_v, dst_v, rows_v, e_v, aggr_sh, sem_g, sem_e):
    cid = lax.axis_index("c")
    sid = lax.axis_index("s")
    wid = sid * NC + cid

    def _zrow(r, carry):
        for k in range(H // 16):
            rows_v[r, pl.ds(k * 16, 16)] = jnp.zeros((16,), jnp.float32)
        return carry
    lax.fori_loop(0, CH, _zrow, 0)
    rows_per_sub = NP // NS
    for t in range(rows_per_sub // CH):
        pltpu.sync_copy(rows_v,
                        aggr_sh.at[pl.ds(sid * rows_per_sub + t * CH, CH)])

    plsc.subcore_barrier()

    ebase2 = wid * (CPW * CH // 2)

    def _group(gi, carry):
        pltpu.sync_copy(src_hbm.at[pl.ds(wid * CPW + gi * 8, 8)], src_v)
        pltpu.sync_copy(dst_hbm.at[pl.ds(wid * CPW + gi * 8, 8)], dst_v)

        def _chunk(jj, carry2):
            j = gi * 8 + jj
            g = pltpu.async_copy(h_hbm.at[src_v.at[jj]], rows_v, sem_g)
            c = pltpu.async_copy(
                e_hbm.at[pl.ds(ebase2 + j * (CH // 2), CH // 2)], e_v, sem_e)
            g.wait()
            c.wait()

            himask = jnp.int32(-65536)

            def _pair(m, cc):
                for half in range(2):
                    r2 = 2 * m + half
                    for k in range(H // 32):
                        ev = e_v[m, pl.ds(half * 64 + k * 16, 16)]
                        ea = lax.bitcast_convert_type(ev << 16, jnp.float32)
                        eb = lax.bitcast_convert_type(ev & himask, jnp.float32)
                        s0 = pl.ds(k * 32, 16)
                        s1 = pl.ds(k * 32 + 16, 16)
                        rows_v[r2, s0] = jnp.maximum(rows_v[r2, s0] + ea, 0.0)
                        rows_v[r2, s1] = jnp.maximum(rows_v[r2, s1] + eb, 0.0)
                return cc
            lax.fori_loop(0, CH // 2, _pair, 0)
            pltpu.sync_copy(rows_v, aggr_sh.at[dst_v.at[jj]], add=True)
            return carry2
        lax.fori_loop(0, 8, _chunk, 0)
        return carry
    lax.fori_loop(0, CPW // 8, _group, 0)

    plsc.subcore_barrier()
    for t in range(rows_per_sub // CH):
        r0 = sid * rows_per_sub + t * CH
        pltpu.sync_copy(aggr_sh.at[pl.ds(r0, CH)],
                        out_hbm.at[cid, pl.ds(r0, CH)])


@functools.partial(
    pl.kernel,
    mesh=_mesh,
    out_type=(jax.ShapeDtypeStruct((NC, GP, H), jnp.float32),
              jax.ShapeDtypeStruct((NC, GP, H), jnp.float32)),
    scratch_types=[
        pltpu.VMEM((CH, H), jnp.float32),
        pltpu.VMEM((CH, H), jnp.float32),
        pltpu.VMEM((80, CH), jnp.int32),
        pltpu.VMEM((16,), jnp.int32),
        pltpu.VMEM_SHARED((GP, H), jnp.float32),
        pltpu.VMEM_SHARED((GP, H), jnp.float32),
    ],
)
def _pool_sc(h_hbm, b2d_hbm, btail_hbm, s_out, c_out,
             hbuf, obuf, bidx2d_v, btail_v, sums_sh, cnts_sh):
    cid = lax.axis_index("c")
    sid = lax.axis_index("s")
    wid = sid * NC + cid
    nfull = (N // CH)

    def _orow(r, carry):
        for k in range(H // 16):
            obuf[r, pl.ds(k * 16, 16)] = jnp.full((16,), 1.0, jnp.float32)
        return carry
    lax.fori_loop(0, CH, _orow, 0)

    def _zrow(r, carry):
        for k in range(H // 16):
            hbuf[r, pl.ds(k * 16, 16)] = jnp.zeros((16,), jnp.float32)
        return carry
    lax.fori_loop(0, GP // NS, _zrow, 0)
    rps = GP // NS
    pltpu.sync_copy(hbuf.at[pl.ds(0, rps)], sums_sh.at[pl.ds(sid * rps, rps)])
    pltpu.sync_copy(hbuf.at[pl.ds(0, rps)], cnts_sh.at[pl.ds(sid * rps, rps)])
    pltpu.sync_copy(b2d_hbm, bidx2d_v)
    plsc.subcore_barrier()

    for t in range((nfull + NW - 1) // NW):
        cidx = wid + NW * t

        @pl.when(cidx < nfull)
        def _():
            pltpu.sync_copy(h_hbm.at[pl.ds(cidx * CH, CH)], hbuf)
            pltpu.sync_copy(hbuf, sums_sh.at[bidx2d_v.at[cidx]], add=True)
            pltpu.sync_copy(obuf, cnts_sh.at[bidx2d_v.at[cidx]], add=True)

    @pl.when(wid == 0)
    def _():
        pltpu.sync_copy(btail_hbm, btail_v)
        pltpu.sync_copy(h_hbm.at[pl.ds(nfull * CH, N - nfull * CH)],
                        hbuf.at[pl.ds(0, N - nfull * CH)])
        pltpu.sync_copy(hbuf.at[pl.ds(0, N - nfull * CH)],
                        sums_sh.at[btail_v], add=True)
        pltpu.sync_copy(obuf.at[pl.ds(0, N - nfull * CH)],
                        cnts_sh.at[btail_v], add=True)

    plsc.subcore_barrier()
    r0 = sid * rps
    pltpu.sync_copy(sums_sh.at[pl.ds(r0, rps)], s_out.at[cid, pl.ds(r0, rps)])
    pltpu.sync_copy(cnts_sh.at[pl.ds(r0, rps)], c_out.at[cid, pl.ds(r0, rps)])


def _proj_node_body(x_ref, w_ref, b_ref, o_ref):
    o_ref[...] = jnp.dot(x_ref[...], w_ref[...],
                         preferred_element_type=jnp.float32) + b_ref[...]


def _proj_edge_body(a_ref, w_ref, b_ref, o_ref):
    o_ref[...] = (jnp.dot(a_ref[...], w_ref[...],
                          preferred_element_type=jnp.float32)
                  + b_ref[...]).astype(jnp.bfloat16)


def _node_body(h_ref, a0_ref, a1_ref, sc_ref, w1_ref, b1_ref,
               w2_ref, b2_ref, g_ref, bt_ref, o_ref):
    h = h_ref[...]
    z = h * sc_ref[...] + (a0_ref[0] + a1_ref[0])
    z = jnp.maximum(jnp.dot(z, w1_ref[...], preferred_element_type=jnp.float32)
                    + b1_ref[...], 0.0)
    z = jnp.dot(z, w2_ref[...], preferred_element_type=jnp.float32) + b2_ref[...]
    z = jnp.maximum(z * g_ref[...] + bt_ref[...], 0.0)
    o_ref[...] = z + h


def _comb_body(s_ref, c_ref, o_ref):
    s = s_ref[0] + s_ref[1]
    cnt = jnp.maximum(c_ref[0] + c_ref[1], 1.0)
    o_ref[...] = (s / cnt)[:G, :]


_NB = 1000


def _node_update(h, aggr, scale_row, w1, b1r, w2, b2r, gr, btr):
    rep = lambda i: (0, 0)
    return pl.pallas_call(
        _node_body,
        grid=(N // _NB,),
        in_specs=[
            pl.BlockSpec((_NB, H), lambda i: (i, 0)),
            pl.BlockSpec((1, _NB, H), lambda i: (0, i, 0)),
            pl.BlockSpec((1, _NB, H), lambda i: (1, i, 0)),
            pl.BlockSpec((1, H), rep),
            pl.BlockSpec((H, H), rep),
            pl.BlockSpec((1, H), rep),
            pl.BlockSpec((H, H), rep),
            pl.BlockSpec((1, H), rep),
            pl.BlockSpec((1, H), rep),
            pl.BlockSpec((1, H), rep),
        ],
        out_specs=pl.BlockSpec((_NB, H), lambda i: (i, 0)),
        out_shape=jax.ShapeDtypeStruct((N, H), jnp.float32),
    )(h, aggr, aggr, scale_row, w1, b1r, w2, b2r, gr, btr)


_EB = 4096


def kernel(x, edge_index, edge_attr, batch, W_np, b_np, W_ep, b_ep, eps,
           W1, b1, W2, b2, gamma, beta):
    f32 = jnp.float32
    src = edge_index[0].astype(jnp.int32)
    dst = edge_index[1].astype(jnp.int32)
    pad = E_PAD - E
    src2d = jnp.concatenate([src, jnp.zeros((pad,), jnp.int32)]).reshape(NW * CPW, CH)
    dst2d = jnp.concatenate([dst, jnp.full((pad,), NP - 1, jnp.int32)]).reshape(NW * CPW, CH)
    ea_pad = jnp.concatenate([edge_attr.astype(f32),
                              jnp.zeros((pad, ED), f32)], axis=0)

    h = pl.pallas_call(
        _proj_node_body,
        out_shape=jax.ShapeDtypeStruct((N, H), f32),
    )(x.astype(f32), W_np.astype(f32), b_np.astype(f32).reshape(1, H))

    e16 = pl.pallas_call(
        _proj_edge_body,
        grid=(E_PAD // _EB,),
        in_specs=[
            pl.BlockSpec((_EB, ED), lambda i: (i, 0)),
            pl.BlockSpec((ED, H), lambda i: (0, 0)),
            pl.BlockSpec((1, H), lambda i: (0, 0)),
        ],
        out_specs=pl.BlockSpec((_EB, H), lambda i: (i, 0)),
        out_shape=jax.ShapeDtypeStruct((E_PAD, H), jnp.bfloat16),
    )(ea_pad, W_ep.astype(f32), b_ep.astype(f32).reshape(1, H))
    e_pair = jnp.transpose(e16.reshape(E_PAD, 4, 2, 16), (0, 1, 3, 2))
    e_packed = lax.bitcast_convert_type(e_pair, jnp.int32).reshape(
        E_PAD // 2, H)

    inv = 1.0 / jnp.sqrt(jnp.float32(1.0 + 1e-5))
    for i in range(L):
        aggr = _edge_sc(h, src2d, dst2d, e_packed)
        scale_row = jnp.full((1, H), 1.0, f32) * (1.0 + eps[i].astype(f32))
        h = _node_update(h, aggr, scale_row,
                         W1[i].astype(f32), b1[i].astype(f32).reshape(1, H),
                         W2[i].astype(f32), b2[i].astype(f32).reshape(1, H),
                         (gamma[i].astype(f32) * inv).reshape(1, H),
                         beta[i].astype(f32).reshape(1, H))

    nfull = N // CH
    b2d = jnp.concatenate(
        [batch[:nfull * CH].astype(jnp.int32),
         jnp.zeros(((80 - nfull) * CH,), jnp.int32)]).reshape(80, CH)
    btail = batch[nfull * CH:].astype(jnp.int32)
    s, c = _pool_sc(h, b2d, btail)

    g = pl.pallas_call(
        _comb_body,
        out_shape=jax.ShapeDtypeStruct((G, H), f32),
    )(s, c)
    return g


# speedup vs baseline: 1.3411x; 1.3411x over previous
import functools

import jax
import jax.numpy as jnp
from jax import lax
from jax.experimental import pallas as pl
from jax.experimental.pallas import tpu as pltpu
from jax.experimental.pallas import tpu_sc as plsc

N = 10000
E = 320000
D = 128
H = 128
ED = 16
L = 5
G = 64

NC = 2
NS = 16
NW = NC * NS
CH = 128
CPW = 80
E_PAD = NW * CPW * CH
NP = 10240
GP = 128

_mesh = plsc.VectorSubcoreMesh(core_axis_name="c", subcore_axis_name="s")


@functools.partial(
    pl.kernel,
    mesh=_mesh,
    out_type=jax.ShapeDtypeStruct((NC, NP, H), jnp.float32),
    scratch_types=[
        pltpu.VMEM((8, CH), jnp.int32),
        pltpu.VMEM((8, CH), jnp.int32),
        pltpu.VMEM((CH, H), jnp.float32),
        pltpu.VMEM((CH, H), jnp.float32),
        pltpu.VMEM_SHARED((NP, H), jnp.float32),
        pltpu.SemaphoreType.DMA,
        pltpu.SemaphoreType.DMA,
    ],
)
def _edge_sc(h_hbm, src_hbm, dst_hbm, e_hbm, out_hbm,
             src_v, dst_v, rows_v, e_v, aggr_sh, sem_g, sem_e):
    cid = lax.axis_index("c")
    sid = lax.axis_index("s")
    wid = sid * NC + cid

    def _zrow(r, carry):
        for k in range(H // 16):
            rows_v[r, pl.ds(k * 16, 16)] = jnp.zeros((16,), jnp.float32)
        return carry
    lax.fori_loop(0, CH, _zrow, 0)
    rows_per_sub = NP // NS
    for t in range(rows_per_sub // CH):
        pltpu.sync_copy(rows_v,
                        aggr_sh.at[pl.ds(sid * rows_per_sub + t * CH, CH)])

    plsc.subcore_barrier()

    ebase = wid * (CPW * CH)

    def _group(gi, carry):
        pltpu.sync_copy(src_hbm.at[pl.ds(wid * CPW + gi * 8, 8)], src_v)
        pltpu.sync_copy(dst_hbm.at[pl.ds(wid * CPW + gi * 8, 8)], dst_v)

        def _chunk(jj, carry2):
            j = gi * 8 + jj
            g = pltpu.async_copy(h_hbm.at[src_v.at[jj]], rows_v, sem_g)
            c = pltpu.async_copy(e_hbm.at[pl.ds(ebase + j * CH, CH)], e_v, sem_e)
            g.wait()
            c.wait()

            @plsc.parallel_loop(0, CH, 1, unroll=4)
            def _row(r):
                for k in range(H // 16):
                    s = pl.ds(k * 16, 16)
                    rows_v[r, s] = jnp.maximum(rows_v[r, s] + e_v[r, s], 0.0)
            pltpu.sync_copy(rows_v, aggr_sh.at[dst_v.at[jj]], add=True)
            return carry2
        lax.fori_loop(0, 8, _chunk, 0)
        return carry
    lax.fori_loop(0, CPW // 8, _group, 0)

    plsc.subcore_barrier()
    for t in range(rows_per_sub // CH):
        r0 = sid * rows_per_sub + t * CH
        pltpu.sync_copy(aggr_sh.at[pl.ds(r0, CH)],
                        out_hbm.at[cid, pl.ds(r0, CH)])


@functools.partial(
    pl.kernel,
    mesh=_mesh,
    out_type=(jax.ShapeDtypeStruct((NC, GP, H), jnp.float32),
              jax.ShapeDtypeStruct((NC, GP, H), jnp.float32)),
    scratch_types=[
        pltpu.VMEM((CH, H), jnp.float32),
        pltpu.VMEM((CH, H), jnp.float32),
        pltpu.VMEM((80, CH), jnp.int32),
        pltpu.VMEM((16,), jnp.int32),
        pltpu.VMEM_SHARED((GP, H), jnp.float32),
        pltpu.VMEM_SHARED((GP, H), jnp.float32),
    ],
)
def _pool_sc(h_hbm, b2d_hbm, btail_hbm, s_out, c_out,
             hbuf, obuf, bidx2d_v, btail_v, sums_sh, cnts_sh):
    cid = lax.axis_index("c")
    sid = lax.axis_index("s")
    wid = sid * NC + cid
    nfull = (N // CH)

    def _orow(r, carry):
        for k in range(H // 16):
            obuf[r, pl.ds(k * 16, 16)] = jnp.full((16,), 1.0, jnp.float32)
        return carry
    lax.fori_loop(0, CH, _orow, 0)

    def _zrow(r, carry):
        for k in range(H // 16):
            hbuf[r, pl.ds(k * 16, 16)] = jnp.zeros((16,), jnp.float32)
        return carry
    lax.fori_loop(0, GP // NS, _zrow, 0)
    rps = GP // NS
    pltpu.sync_copy(hbuf.at[pl.ds(0, rps)], sums_sh.at[pl.ds(sid * rps, rps)])
    pltpu.sync_copy(hbuf.at[pl.ds(0, rps)], cnts_sh.at[pl.ds(sid * rps, rps)])
    pltpu.sync_copy(b2d_hbm, bidx2d_v)
    plsc.subcore_barrier()

    for t in range((nfull + NW - 1) // NW):
        cidx = wid + NW * t

        @pl.when(cidx < nfull)
        def _():
            pltpu.sync_copy(h_hbm.at[pl.ds(cidx * CH, CH)], hbuf)
            pltpu.sync_copy(hbuf, sums_sh.at[bidx2d_v.at[cidx]], add=True)
            pltpu.sync_copy(obuf, cnts_sh.at[bidx2d_v.at[cidx]], add=True)

    @pl.when(wid == 0)
    def _():
        pltpu.sync_copy(btail_hbm, btail_v)
        pltpu.sync_copy(h_hbm.at[pl.ds(nfull * CH, N - nfull * CH)],
                        hbuf.at[pl.ds(0, N - nfull * CH)])
        pltpu.sync_copy(hbuf.at[pl.ds(0, N - nfull * CH)],
                        sums_sh.at[btail_v], add=True)
        pltpu.sync_copy(obuf.at[pl.ds(0, N - nfull * CH)],
                        cnts_sh.at[btail_v], add=True)

    plsc.subcore_barrier()
    r0 = sid * rps
    pltpu.sync_copy(sums_sh.at[pl.ds(r0, rps)], s_out.at[cid, pl.ds(r0, rps)])
    pltpu.sync_copy(cnts_sh.at[pl.ds(r0, rps)], c_out.at[cid, pl.ds(r0, rps)])


def _proj_node_body(x_ref, w_ref, b_ref, o_ref):
    o_ref[...] = jnp.dot(x_ref[...], w_ref[...],
                         preferred_element_type=jnp.float32) + b_ref[...]


def _proj_edge_body(a_ref, w_ref, b_ref, o_ref):
    o_ref[...] = jnp.dot(a_ref[...], w_ref[...],
                         preferred_element_type=jnp.float32) + b_ref[...]


def _node_body(h_ref, a0_ref, a1_ref, sc_ref, w1_ref, b1_ref,
               w2_ref, b2_ref, g_ref, bt_ref, o_ref):
    h = h_ref[...]
    z = h * sc_ref[...] + (a0_ref[0] + a1_ref[0])
    z = jnp.maximum(jnp.dot(z, w1_ref[...], preferred_element_type=jnp.float32)
                    + b1_ref[...], 0.0)
    z = jnp.dot(z, w2_ref[...], preferred_element_type=jnp.float32) + b2_ref[...]
    z = jnp.maximum(z * g_ref[...] + bt_ref[...], 0.0)
    o_ref[...] = z + h


def _comb_body(s_ref, c_ref, o_ref):
    s = s_ref[0] + s_ref[1]
    cnt = jnp.maximum(c_ref[0] + c_ref[1], 1.0)
    o_ref[...] = (s / cnt)[:G, :]


_NB = 1000


def _node_update(h, aggr, scale_row, w1, b1r, w2, b2r, gr, btr):
    rep = lambda i: (0, 0)
    return pl.pallas_call(
        _node_body,
        grid=(N // _NB,),
        in_specs=[
            pl.BlockSpec((_NB, H), lambda i: (i, 0)),
            pl.BlockSpec((1, _NB, H), lambda i: (0, i, 0)),
            pl.BlockSpec((1, _NB, H), lambda i: (1, i, 0)),
            pl.BlockSpec((1, H), rep),
            pl.BlockSpec((H, H), rep),
            pl.BlockSpec((1, H), rep),
            pl.BlockSpec((H, H), rep),
            pl.BlockSpec((1, H), rep),
            pl.BlockSpec((1, H), rep),
            pl.BlockSpec((1, H), rep),
        ],
        out_specs=pl.BlockSpec((_NB, H), lambda i: (i, 0)),
        out_shape=jax.ShapeDtypeStruct((N, H), jnp.float32),
    )(h, aggr, aggr, scale_row, w1, b1r, w2, b2r, gr, btr)


_EB = 4096


def kernel(x, edge_index, edge_attr, batch, W_np, b_np, W_ep, b_ep, eps,
           W1, b1, W2, b2, gamma, beta):
    f32 = jnp.float32
    src = edge_index[0].astype(jnp.int32)
    dst = edge_index[1].astype(jnp.int32)
    pad = E_PAD - E
    src2d = jnp.concatenate([src, jnp.zeros((pad,), jnp.int32)]).reshape(NW * CPW, CH)
    pad_dst = N + jnp.arange(pad, dtype=jnp.int32) % (NP - N)
    dst2d = jnp.concatenate([dst, pad_dst]).reshape(NW * CPW, CH)
    ea_pad = jnp.concatenate([edge_attr.astype(f32),
                              jnp.zeros((pad, ED), f32)], axis=0)

    h = pl.pallas_call(
        _proj_node_body,
        out_shape=jax.ShapeDtypeStruct((N, H), f32),
    )(x.astype(f32), W_np.astype(f32), b_np.astype(f32).reshape(1, H))

    e = pl.pallas_call(
        _proj_edge_body,
        grid=(E_PAD // _EB,),
        in_specs=[
            pl.BlockSpec((_EB, ED), lambda i: (i, 0)),
            pl.BlockSpec((ED, H), lambda i: (0, 0)),
            pl.BlockSpec((1, H), lambda i: (0, 0)),
        ],
        out_specs=pl.BlockSpec((_EB, H), lambda i: (i, 0)),
        out_shape=jax.ShapeDtypeStruct((E_PAD, H), f32),
    )(ea_pad, W_ep.astype(f32), b_ep.astype(f32).reshape(1, H))

    inv = 1.0 / jnp.sqrt(jnp.float32(1.0 + 1e-5))
    for i in range(L):
        aggr = _edge_sc(h, src2d, dst2d, e)
        scale_row = jnp.full((1, H), 1.0, f32) * (1.0 + eps[i].astype(f32))
        h = _node_update(h, aggr, scale_row,
                         W1[i].astype(f32), b1[i].astype(f32).reshape(1, H),
                         W2[i].astype(f32), b2[i].astype(f32).reshape(1, H),
                         (gamma[i].astype(f32) * inv).reshape(1, H),
                         beta[i].astype(f32).reshape(1, H))

    nfull = N // CH
    b2d = jnp.concatenate(
        [batch[:nfull * CH].astype(jnp.int32),
         jnp.zeros(((80 - nfull) * CH,), jnp.int32)]).reshape(80, CH)
    btail = batch[nfull * CH:].astype(jnp.int32)
    s, c = _pool_sc(h, b2d, btail)

    g = pl.pallas_call(
        _comb_body,
        out_shape=jax.ShapeDtypeStruct((G, H), f32),
    )(s, c)
    return g
